# Initial kernel scaffold; baseline (speedup 1.0000x reference)
import functools

import jax
import jax.numpy as jnp
from jax import lax
from jax.experimental import pallas as pl
from jax.experimental.pallas import tpu as pltpu
from jax.experimental.pallas import tpu_sc as plsc

VOCAB = 100000
EMBED_DIM = 64
BATCH = 4096
SEQ_LEN = 50

NUM_WORKERS = 32
TOTAL = BATCH * SEQ_LEN
PER_W = TOTAL // NUM_WORKERS
CHUNK = 128
NCHUNK = PER_W // CHUNK

_MESH = plsc.VectorSubcoreMesh(core_axis_name="c", subcore_axis_name="s")


@functools.partial(
    pl.kernel,
    out_type=jax.ShapeDtypeStruct((NUM_WORKERS, NCHUNK, CHUNK, EMBED_DIM),
                                  jnp.float32),
    mesh=_MESH,
    scratch_types=[
        pltpu.VMEM((NCHUNK, CHUNK), jnp.int32),
        pltpu.VMEM((CHUNK, EMBED_DIM), jnp.float32),
        pltpu.SemaphoreType.DMA,
    ],
)
def _gather_kernel(idx_hbm, table_hbm, out_hbm, idx_v, rows_v, gsem):
    wid = lax.axis_index("s") * 2 + lax.axis_index("c")
    pltpu.sync_copy(idx_hbm.at[wid], idx_v)

    def chunk_body(j):
        pltpu.async_copy(table_hbm.at[idx_v.at[j]], rows_v, gsem).wait()
        pltpu.sync_copy(rows_v, out_hbm.at[wid, j])

    pl.loop(0, NCHUNK)(chunk_body)


def kernel(inputs, table):
    idx = inputs.reshape(NUM_WORKERS, NCHUNK, CHUNK)
    out = _gather_kernel(idx, table)
    return out.reshape(BATCH, SEQ_LEN, EMBED_DIM)


# speedup vs baseline: 4.0971x; 4.0971x over previous
import functools

import jax
import jax.numpy as jnp
from jax import lax
from jax.experimental import pallas as pl
from jax.experimental.pallas import tpu as pltpu
from jax.experimental.pallas import tpu_sc as plsc

VOCAB = 100000
EMBED_DIM = 64
BATCH = 4096
SEQ_LEN = 50

NUM_WORKERS = 32
TOTAL = BATCH * SEQ_LEN
PER_W = TOTAL // NUM_WORKERS
CHUNK = 128
NCHUNK = PER_W // CHUNK

_MESH = plsc.VectorSubcoreMesh(core_axis_name="c", subcore_axis_name="s")


@functools.partial(
    pl.kernel,
    out_type=jax.ShapeDtypeStruct((NUM_WORKERS, NCHUNK, CHUNK, EMBED_DIM),
                                  jnp.float32),
    mesh=_MESH,
    scratch_types=[
        pltpu.VMEM((NCHUNK, CHUNK), jnp.int32),
        pltpu.VMEM((CHUNK, EMBED_DIM), jnp.float32),
        pltpu.SemaphoreType.DMA,
    ],
    compiler_params=pltpu.CompilerParams(use_tc_tiling_on_sc=False),
)
def _gather_kernel(idx_hbm, table_hbm, out_hbm, idx_v, rows_v, gsem):
    wid = lax.axis_index("s") * 2 + lax.axis_index("c")
    pltpu.sync_copy(idx_hbm.at[wid], idx_v)

    def chunk_body(j):
        pltpu.async_copy(table_hbm.at[idx_v.at[j]], rows_v, gsem).wait()
        pltpu.sync_copy(rows_v, out_hbm.at[wid, j])

    pl.loop(0, NCHUNK)(chunk_body)


def kernel(inputs, table):
    idx = inputs.reshape(NUM_WORKERS, NCHUNK, CHUNK)
    out = _gather_kernel(idx, table)
    return out.reshape(BATCH, SEQ_LEN, EMBED_DIM)


# speedup vs baseline: 4.6621x; 1.1379x over previous
import functools

import jax
import jax.numpy as jnp
from jax import lax
from jax.experimental import pallas as pl
from jax.experimental.pallas import tpu as pltpu
from jax.experimental.pallas import tpu_sc as plsc

VOCAB = 100000
EMBED_DIM = 64
BATCH = 4096
SEQ_LEN = 50

NUM_WORKERS = 32
TOTAL = BATCH * SEQ_LEN
PER_W = TOTAL // NUM_WORKERS
CHUNK = 128
NCHUNK = PER_W // CHUNK
NBUF = 5
NGROUP = NCHUNK // NBUF

_MESH = plsc.VectorSubcoreMesh(core_axis_name="c", subcore_axis_name="s")


@functools.partial(
    pl.kernel,
    out_type=jax.ShapeDtypeStruct((NUM_WORKERS, NCHUNK, CHUNK, EMBED_DIM),
                                  jnp.float32),
    mesh=_MESH,
    scratch_types=[
        pltpu.VMEM((NCHUNK, CHUNK), jnp.int32),
        pltpu.VMEM((NBUF, CHUNK, EMBED_DIM), jnp.float32),
    ]
    + [pltpu.SemaphoreType.DMA] * (2 * NBUF),
    compiler_params=pltpu.CompilerParams(use_tc_tiling_on_sc=False),
)
def _gather_kernel(idx_hbm, table_hbm, out_hbm, idx_v, rows_v, *sems):
    gsems, ssems = sems[:NBUF], sems[NBUF:]
    wid = lax.axis_index("s") * 2 + lax.axis_index("c")
    pltpu.sync_copy(idx_hbm.at[wid], idx_v)

    def g_copy(j, b):
        return pltpu.make_async_copy(
            table_hbm.at[idx_v.at[j]], rows_v.at[b], gsems[b])

    def s_copy(j, b):
        return pltpu.make_async_copy(
            rows_v.at[b], out_hbm.at[wid, j], ssems[b])

    for b in range(NBUF):
        g_copy(b, b).start()

    def group_body(g):
        j0 = g * NBUF
        for b in range(NBUF):
            g_copy(j0 + b, b).wait()
            s_copy(j0 + b, b).start()
        for b in range(NBUF):
            s_copy(j0 + b, b).wait()
            g_copy(j0 + NBUF + b, b).start()

    pl.loop(0, NGROUP - 1)(group_body)

    j0 = NCHUNK - NBUF
    for b in range(NBUF):
        g_copy(j0 + b, b).wait()
        s_copy(j0 + b, b).start()
    for b in range(NBUF):
        s_copy(j0 + b, b).wait()


def kernel(inputs, table):
    idx = inputs.reshape(NUM_WORKERS, NCHUNK, CHUNK)
    out = _gather_kernel(idx, table)
    return out.reshape(BATCH, SEQ_LEN, EMBED_DIM)
